# baseline (speedup 1.0000x reference)
import functools

import jax
import jax.numpy as jnp
from jax import lax
from jax.experimental import pallas as pl
from jax.experimental.pallas import tpu as pltpu
from jax.experimental.pallas import tpu_sc as plsc

EMBED = 64
MAX_ATOMS = 15
NUM_AT = 6
NUM_AP = 11
NUM_COMB = NUM_AT * NUM_AP
RES_CHUNK = 128
NTR = 3
TRS_PER_CHUNK = MAX_ATOMS
BATCHES = TRS_PER_CHUNK // NTR
GROUPS = NTR * 8
RFW = RES_CHUNK * EMBED
ATW = RES_CHUNK * 16
MW = 256
OBW = NTR * 8 * 128
OBATCH = 8 * OBW

_GDN = lax.GatherDimensionNumbers(
    offset_dims=(), collapsed_slice_dims=(0,), start_index_map=(0,))


def _make_sc_call(R):
    info = plsc.get_sparse_core_info()
    NC, NS = info.num_cores, info.num_subcores
    NW = NC * NS
    per_w = R // NW
    n_chunks = per_w // RES_CHUNK
    n_pairs = n_chunks // 2
    NROWS = R * MAX_ATOMS
    TQ = (NROWS // 128) * 1024
    mesh = plsc.VectorSubcoreMesh(core_axis_name="c", subcore_axis_name="s")

    @functools.partial(
        pl.kernel,
        mesh=mesh,
        compiler_params=pltpu.CompilerParams(needs_layout_passes=False),
        out_type=jax.ShapeDtypeStruct((NROWS * EMBED,), jnp.float32),
        scratch_types=[
            pltpu.VMEM(((NUM_COMB + 1) * EMBED,), jnp.float32),
            pltpu.VMEM((2 * RFW,), jnp.float32),
            pltpu.VMEM((2 * ATW,), jnp.int32),
            pltpu.VMEM((2 * MW,), jnp.float32),
            pltpu.VMEM((RFW,), jnp.float32),
            pltpu.VMEM((2 * OBATCH,), jnp.float32),
            pltpu.SemaphoreType.DMA,
            pltpu.SemaphoreType.DMA,
            pltpu.SemaphoreType.DMA,
            pltpu.SemaphoreType.DMA,
        ],
    )
    def sc_call(rf_hbm, atp_hbm, m_hbm, atab_hbm, ptab_hbm, out_hbm,
                comb_v, rf_v, atp_v, m_v, rfm_v, out_v,
                isem0, isem1, osem0, osem1):
        isems = (isem0, isem1)
        osems = (osem0, osem1)
        wid = lax.axis_index("s") * NC + lax.axis_index("c")
        rbase = wid * per_w
        trbase = wid * (per_w * MAX_ATOMS // 128)

        pltpu.sync_copy(atab_hbm, rfm_v.at[pl.ds(0, NUM_AT * EMBED)])
        pltpu.sync_copy(
            ptab_hbm, rfm_v.at[pl.ds(NUM_AT * EMBED, NUM_AP * EMBED)])
        arow = [[rfm_v[pl.ds(a * EMBED + c * 16, 16)] for c in range(4)]
                for a in range(NUM_AT)]
        prow = [[rfm_v[pl.ds((NUM_AT + p) * EMBED + c * 16, 16)]
                 for c in range(4)] for p in range(NUM_AP)]
        for a in range(NUM_AT):
            for p in range(NUM_AP):
                row = a * NUM_AP + p
                for c in range(4):
                    comb_v[pl.ds(row * EMBED + c * 16, 16)] = (
                        arow[a][c] + prow[p][c])
        for c in range(4):
            comb_v[pl.ds(NUM_COMB * EMBED + c * 16, 16)] = jnp.zeros(
                (16,), jnp.float32)

        iota = lax.iota(jnp.int32, 16)

        def start_in(k, b):
            r0 = rbase + k * RES_CHUNK
            pltpu.async_copy(rf_hbm.at[pl.ds(r0 * EMBED, RFW)],
                             rf_v.at[pl.ds(b * RFW, RFW)], isems[b])
            pltpu.async_copy(atp_hbm.at[pl.ds(r0 * 16, ATW)],
                             atp_v.at[pl.ds(b * ATW, ATW)], isems[b])
            pltpu.async_copy(m_hbm.at[pl.ds(r0, RES_CHUNK)],
                             m_v.at[pl.ds(b * MW, RES_CHUNK)], isems[b])

        def wait_in(b):
            pltpu.make_async_copy(rf_hbm.at[pl.ds(0, RFW)],
                                  rf_v.at[pl.ds(b * RFW, RFW)],
                                  isems[b]).wait()
            pltpu.make_async_copy(atp_hbm.at[pl.ds(0, ATW)],
                                  atp_v.at[pl.ds(b * ATW, ATW)],
                                  isems[b]).wait()
            pltpu.make_async_copy(m_hbm.at[pl.ds(0, RES_CHUNK)],
                                  m_v.at[pl.ds(b * MW, RES_CHUNK)],
                                  isems[b]).wait()

        def start_out(k, t, par, sem):
            tr0 = trbase + k * TRS_PER_CHUNK + t * NTR
            for tc in range(8):
                pltpu.async_copy(
                    out_v.at[pl.ds(par * OBATCH + tc * OBW, OBW)],
                    out_hbm.at[pl.ds(tc * TQ + tr0 * 1024, OBW)], sem)

        def wait_out(sem):
            pltpu.make_async_copy(
                out_hbm.at[pl.ds(0, OBATCH)],
                out_v.at[pl.ds(0, OBATCH)], sem).wait()

        def prep_rfm(b):
            @plsc.parallel_loop(0, RES_CHUNK, unroll=2)
            def _prep(r):
                mvf = plsc.load_gather(
                    m_v, [jnp.full((16,), b * MW + r, jnp.int32)])
                for c in range(4):
                    rfm_v[pl.ds(r * EMBED + c * 16, 16)] = (
                        rf_v[pl.ds(b * RFW + r * EMBED + c * 16, 16)] * mvf)

        def compute_batch(b, t, par):
            @plsc.parallel_loop(0, GROUPS, unroll=4)
            def _body(g):
                abase = t * (NTR * 128) + g * 16
                av = jnp.full((16,), abase, jnp.int32) + iota
                resv = lax.shift_right_logical(av * 2185, 15)
                aidx = av + resv + (b * ATW)
                atpv = plsc.load_gather(atp_v, [aidx])
                atv = lax.shift_right_logical(atpv, 4)
                apv = atpv & 15
                mv = plsc.load_gather(m_v, [resv + (b * MW)])
                civ64 = jnp.where(
                    mv != 0.0, atv * NUM_AP + apv,
                    jnp.full((16,), NUM_COMB, jnp.int32)) * EMBED
                res64 = resv * EMBED
                off0 = par * OBATCH + (g >> 3) * 1024 + (g & 7) * 16
                for c in range(EMBED):
                    gc = plsc.load_gather(comb_v, [civ64 + c])
                    gr = plsc.load_gather(rfm_v, [res64 + c])
                    out_v[pl.ds(off0 + (c // 8) * OBW + (c % 8) * 128,
                                16)] = gc + gr

        start_in(0, 0)
        start_in(1, 1)

        def pair_body(kp, _):
            for b in range(2):
                k = kp * 2 + b
                wait_in(b)
                prep_rfm(b)

                def t_body(t, _):
                    gslot = b * BATCHES + t
                    par = gslot & 1

                    @pl.when((kp > 0) | (gslot >= 2))
                    def _():
                        @pl.when(par == 0)
                        def _():
                            wait_out(osem0)

                        @pl.when(par == 1)
                        def _():
                            wait_out(osem1)

                    compute_batch(b, t, par)

                    @pl.when(par == 0)
                    def _():
                        start_out(k, t, par, osem0)

                    @pl.when(par == 1)
                    def _():
                        start_out(k, t, par, osem1)
                    return 0

                lax.fori_loop(0, BATCHES, t_body, 0)

                @pl.when(k + 2 < n_chunks)
                def _():
                    start_in(k + 2, b)
            return 0

        lax.fori_loop(0, n_pairs, pair_body, 0)
        wait_out(osem0)
        wait_out(osem1)

    return sc_call


def kernel(res_feat, atom_types, atom_positions, mask_atoms, block_lengths,
           atom_table, pos_table):
    N, L, E = res_feat.shape
    A = atom_types.shape[-1]
    R = N * L
    rf1 = res_feat.reshape(R * E)
    atp = (atom_types.astype(jnp.int32) * 16
           + atom_positions.astype(jnp.int32)).reshape(R * A)
    mf = mask_atoms[:, :, 1].reshape(R).astype(jnp.float32)
    atab1 = atom_table.reshape(NUM_AT * EMBED)
    ptab1 = pos_table.reshape(NUM_AP * EMBED)
    sc_call = _make_sc_call(R)
    out = sc_call(rf1, atp, mf, atab1, ptab1)
    nrow = R * MAX_ATOMS
    return (out.reshape(EMBED // 8, nrow // 128, 8, 128)
            .transpose(1, 3, 0, 2).reshape(nrow, EMBED))


# speedup vs baseline: 4.0407x; 4.0407x over previous
import functools

import jax
import jax.numpy as jnp
from jax import lax
from jax.experimental import pallas as pl
from jax.experimental.pallas import tpu as pltpu
from jax.experimental.pallas import tpu_sc as plsc

EMBED = 64
MAX_ATOMS = 15
NUM_AT = 6
NUM_AP = 11
NUM_COMB = NUM_AT * NUM_AP
RES_CHUNK = 128
NTR = 3
TRS_PER_CHUNK = MAX_ATOMS
BATCHES = TRS_PER_CHUNK // NTR
GROUPS = NTR * 8
RFW = RES_CHUNK * EMBED
ATW = RES_CHUNK * 16
MW = 256
OBW = NTR * 8 * 128
OBATCH = 8 * OBW
CSTRIDE = NUM_COMB + 1
RSTRIDE = RES_CHUNK + 1

_GDN = lax.GatherDimensionNumbers(
    offset_dims=(), collapsed_slice_dims=(0,), start_index_map=(0,))


def _make_sc_call(R):
    info = plsc.get_sparse_core_info()
    NC, NS = info.num_cores, info.num_subcores
    NW = NC * NS
    per_w = R // NW
    n_chunks = per_w // RES_CHUNK
    n_pairs = n_chunks // 2
    NROWS = R * MAX_ATOMS
    TQ = (NROWS // 128) * 1024
    mesh = plsc.VectorSubcoreMesh(core_axis_name="c", subcore_axis_name="s")

    @functools.partial(
        pl.kernel,
        mesh=mesh,
        compiler_params=pltpu.CompilerParams(needs_layout_passes=False),
        out_type=jax.ShapeDtypeStruct((NROWS * EMBED,), jnp.float32),
        scratch_types=[
            pltpu.VMEM(((NUM_COMB + 1) * EMBED,), jnp.float32),
            pltpu.VMEM((2 * RFW,), jnp.float32),
            pltpu.VMEM((2 * ATW,), jnp.int32),
            pltpu.VMEM((2 * MW,), jnp.float32),
            pltpu.VMEM((EMBED * RSTRIDE,), jnp.float32),
            pltpu.VMEM((2 * OBATCH,), jnp.float32),
            pltpu.SemaphoreType.DMA,
            pltpu.SemaphoreType.DMA,
            pltpu.SemaphoreType.DMA,
            pltpu.SemaphoreType.DMA,
        ],
    )
    def sc_call(rf_hbm, atp_hbm, m_hbm, atab_hbm, ptab_hbm, out_hbm,
                comb_v, rf_v, atp_v, m_v, rfm_v, out_v,
                isem0, isem1, osem0, osem1):
        isems = (isem0, isem1)
        osems = (osem0, osem1)
        wid = lax.axis_index("s") * NC + lax.axis_index("c")
        rbase = wid * per_w
        trbase = wid * (per_w * MAX_ATOMS // 128)

        iota = lax.iota(jnp.int32, 16)
        pltpu.sync_copy(atab_hbm, rfm_v.at[pl.ds(0, NUM_AT * EMBED)])
        pltpu.sync_copy(
            ptab_hbm, rfm_v.at[pl.ds(NUM_AT * EMBED, NUM_AP * EMBED)])
        arow = [[rfm_v[pl.ds(a * EMBED + c * 16, 16)] for c in range(4)]
                for a in range(NUM_AT)]
        prow = [[rfm_v[pl.ds((NUM_AT + p) * EMBED + c * 16, 16)]
                 for c in range(4)] for p in range(NUM_AP)]
        cidx = [(iota + c0 * 16) * CSTRIDE for c0 in range(4)]
        for a in range(NUM_AT):
            for p in range(NUM_AP):
                row = a * NUM_AP + p
                for c in range(4):
                    plsc.store_scatter(comb_v, [cidx[c] + row],
                                       arow[a][c] + prow[p][c])
        for c in range(4):
            plsc.store_scatter(comb_v, [cidx[c] + NUM_COMB],
                               jnp.zeros((16,), jnp.float32))
        cridx = [(iota + c0 * 16) * RSTRIDE for c0 in range(4)]

        def start_in(k, b):
            r0 = rbase + k * RES_CHUNK
            pltpu.async_copy(rf_hbm.at[pl.ds(r0 * EMBED, RFW)],
                             rf_v.at[pl.ds(b * RFW, RFW)], isems[b])
            pltpu.async_copy(atp_hbm.at[pl.ds(r0 * 16, ATW)],
                             atp_v.at[pl.ds(b * ATW, ATW)], isems[b])
            pltpu.async_copy(m_hbm.at[pl.ds(r0, RES_CHUNK)],
                             m_v.at[pl.ds(b * MW, RES_CHUNK)], isems[b])

        def wait_in(b):
            pltpu.make_async_copy(rf_hbm.at[pl.ds(0, RFW)],
                                  rf_v.at[pl.ds(b * RFW, RFW)],
                                  isems[b]).wait()
            pltpu.make_async_copy(atp_hbm.at[pl.ds(0, ATW)],
                                  atp_v.at[pl.ds(b * ATW, ATW)],
                                  isems[b]).wait()
            pltpu.make_async_copy(m_hbm.at[pl.ds(0, RES_CHUNK)],
                                  m_v.at[pl.ds(b * MW, RES_CHUNK)],
                                  isems[b]).wait()

        def start_out(k, t, par, sem):
            tr0 = trbase + k * TRS_PER_CHUNK + t * NTR
            for tc in range(8):
                pltpu.async_copy(
                    out_v.at[pl.ds(par * OBATCH + tc * OBW, OBW)],
                    out_hbm.at[pl.ds(tc * TQ + tr0 * 1024, OBW)], sem)

        def wait_out(sem):
            pltpu.make_async_copy(
                out_hbm.at[pl.ds(0, OBATCH)],
                out_v.at[pl.ds(0, OBATCH)], sem).wait()

        def prep_rfm(b):
            @plsc.parallel_loop(0, RES_CHUNK, unroll=2)
            def _prep(r):
                mvf = plsc.load_gather(
                    m_v, [jnp.full((16,), b * MW + r, jnp.int32)])
                for c in range(4):
                    plsc.store_scatter(
                        rfm_v, [cridx[c] + r],
                        rf_v[pl.ds(b * RFW + r * EMBED + c * 16, 16)] * mvf)

        def compute_batch(b, t, par):
            @plsc.parallel_loop(0, GROUPS, unroll=4)
            def _body(g):
                abase = t * (NTR * 128) + g * 16
                av = jnp.full((16,), abase, jnp.int32) + iota
                resv = lax.shift_right_logical(av * 2185, 15)
                aidx = av + resv + (b * ATW)
                atpv = plsc.load_gather(atp_v, [aidx])
                atv = lax.shift_right_logical(atpv, 4)
                apv = atpv & 15
                mv = plsc.load_gather(m_v, [resv + (b * MW)])
                civ = jnp.where(
                    mv != 0.0, atv * NUM_AP + apv,
                    jnp.full((16,), NUM_COMB, jnp.int32))
                off0 = par * OBATCH + (g >> 3) * 1024 + (g & 7) * 16
                for c in range(EMBED):
                    gc = plsc.load_gather(comb_v, [civ + c * CSTRIDE])
                    gr = plsc.load_gather(rfm_v, [resv + c * RSTRIDE])
                    out_v[pl.ds(off0 + (c // 8) * OBW + (c % 8) * 128,
                                16)] = gc + gr

        start_in(0, 0)
        start_in(1, 1)

        def pair_body(kp, _):
            for b in range(2):
                k = kp * 2 + b
                wait_in(b)
                prep_rfm(b)

                def t_body(t, _):
                    gslot = b * BATCHES + t
                    par = gslot & 1

                    @pl.when((kp > 0) | (gslot >= 2))
                    def _():
                        @pl.when(par == 0)
                        def _():
                            wait_out(osem0)

                        @pl.when(par == 1)
                        def _():
                            wait_out(osem1)

                    compute_batch(b, t, par)

                    @pl.when(par == 0)
                    def _():
                        start_out(k, t, par, osem0)

                    @pl.when(par == 1)
                    def _():
                        start_out(k, t, par, osem1)
                    return 0

                lax.fori_loop(0, BATCHES, t_body, 0)

                @pl.when(k + 2 < n_chunks)
                def _():
                    start_in(k + 2, b)
            return 0

        lax.fori_loop(0, n_pairs, pair_body, 0)
        wait_out(osem0)
        wait_out(osem1)

    return sc_call


def kernel(res_feat, atom_types, atom_positions, mask_atoms, block_lengths,
           atom_table, pos_table):
    N, L, E = res_feat.shape
    A = atom_types.shape[-1]
    R = N * L
    rf1 = res_feat.reshape(R * E)
    atp = (atom_types.astype(jnp.int32) * 16
           + atom_positions.astype(jnp.int32)).reshape(R * A)
    mf = mask_atoms[:, :, 1].reshape(R).astype(jnp.float32)
    atab1 = atom_table.reshape(NUM_AT * EMBED)
    ptab1 = pos_table.reshape(NUM_AP * EMBED)
    sc_call = _make_sc_call(R)
    out = sc_call(rf1, atp, mf, atab1, ptab1)
    nrow = R * MAX_ATOMS
    return (out.reshape(EMBED // 8, nrow // 128, 8, 128)
            .transpose(1, 3, 0, 2).reshape(nrow, EMBED))
